# baseline (speedup 1.0000x reference)
import functools

import jax
import jax.numpy as jnp
from jax import lax
from jax.experimental import pallas as pl
from jax.experimental.pallas import tpu as pltpu
from jax.experimental.pallas import tpu_sc as plsc

_B = 16384
_D = 32
_H = 64
_CHUNK = 128

_info = plsc.get_sparse_core_info()
_NC, _NS = _info.num_cores, _info.num_subcores
_NW = _NC * _NS
_BPW = _B // _NW
_CPW = _BPW // _CHUNK

_mesh = plsc.VectorSubcoreMesh(core_axis_name="c", subcore_axis_name="s")


@functools.partial(
    pl.kernel,
    mesh=_mesh,
    compiler_params=pltpu.CompilerParams(use_tc_tiling_on_sc=False),
    out_type=(
        jax.ShapeDtypeStruct((_B, _D), jnp.float32),
        jax.ShapeDtypeStruct((_B, _D), jnp.float32),
    ),
    scratch_types=[
        pltpu.VMEM((_CPW, _CHUNK), jnp.int32),
        pltpu.VMEM((_CPW, _CHUNK), jnp.int32),
        pltpu.VMEM((_BPW, _D), jnp.float32),
        pltpu.VMEM((_BPW, _D), jnp.float32),
        pltpu.SemaphoreType.DMA,
    ],
)
def _gather_sc(uidx_hbm, iidx_hbm, utab_hbm, itab_hbm, u_out, i_out,
               uidx_v, iidx_v, urows_v, irows_v, sem):
    wid = lax.axis_index("s") * _NC + lax.axis_index("c")
    base = wid * _BPW
    pltpu.sync_copy(uidx_hbm.at[pl.ds(wid * _CPW, _CPW)], uidx_v)
    pltpu.sync_copy(iidx_hbm.at[pl.ds(wid * _CPW, _CPW)], iidx_v)
    copies = []
    for j in range(_CPW):
        copies.append(pltpu.async_copy(
            utab_hbm.at[uidx_v.at[j]],
            urows_v.at[pl.ds(j * _CHUNK, _CHUNK)], sem))
        copies.append(pltpu.async_copy(
            itab_hbm.at[iidx_v.at[j]],
            irows_v.at[pl.ds(j * _CHUNK, _CHUNK)], sem))
    for c in copies:
        c.wait()
    pltpu.sync_copy(urows_v, u_out.at[pl.ds(base, _BPW)])
    pltpu.sync_copy(irows_v, i_out.at[pl.ds(base, _BPW)])


def _mlp_tc(u_ref, i_ref, w1_ref, b1_ref, w2_ref, b2_ref, o_ref):
    h = jnp.dot(u_ref[...], w1_ref[0:_D, :], preferred_element_type=jnp.float32)
    h = h + jnp.dot(i_ref[...], w1_ref[_D:, :], preferred_element_type=jnp.float32)
    h = jnp.maximum(h + b1_ref[...], 0.0)
    logits = jnp.dot(h, w2_ref[...], preferred_element_type=jnp.float32)
    o_ref[...] = jax.nn.sigmoid(logits + b2_ref[...])


_BLK = 2048


def kernel(user_indices, item_indices, user_table, item_table, W1, b1, W2, b2):
    uidx = user_indices.astype(jnp.int32).reshape(_NW * _CPW, _CHUNK)
    iidx = item_indices.astype(jnp.int32).reshape(_NW * _CPW, _CHUNK)
    u_emb, i_emb = _gather_sc(uidx, iidx, user_table, item_table)

    out = pl.pallas_call(
        _mlp_tc,
        grid=(_B // _BLK,),
        in_specs=[
            pl.BlockSpec((_BLK, _D), lambda b: (b, 0)),
            pl.BlockSpec((_BLK, _D), lambda b: (b, 0)),
            pl.BlockSpec((2 * _D, _H), lambda b: (0, 0)),
            pl.BlockSpec((1, _H), lambda b: (0, 0)),
            pl.BlockSpec((_H, 1), lambda b: (0, 0)),
            pl.BlockSpec((1, 1), lambda b: (0, 0)),
        ],
        out_specs=pl.BlockSpec((_BLK, 1), lambda b: (b, 0)),
        out_shape=jax.ShapeDtypeStruct((_B, 1), jnp.float32),
    )(u_emb, i_emb, W1, b1.reshape(1, _H), W2, b2.reshape(1, 1))
    return out


# speedup vs baseline: 1.5598x; 1.5598x over previous
import functools

import jax
import jax.numpy as jnp
from jax import lax
from jax.experimental import pallas as pl
from jax.experimental.pallas import tpu as pltpu
from jax.experimental.pallas import tpu_sc as plsc

_B = 16384
_D = 32
_H = 64

_info = plsc.get_sparse_core_info()
_NC, _NS = _info.num_cores, _info.num_subcores
_NW = _NC * _NS
_BPW = _B // _NW

_mesh = plsc.VectorSubcoreMesh(core_axis_name="c", subcore_axis_name="s")


@functools.partial(
    pl.kernel,
    mesh=_mesh,
    out_type=(
        jax.ShapeDtypeStruct((_B, _D), jnp.float32),
        jax.ShapeDtypeStruct((_B, _D), jnp.float32),
    ),
    scratch_types=[
        pltpu.SMEM((_BPW,), jnp.int32),
        pltpu.VMEM((_BPW,), jnp.int32),
        pltpu.VMEM((_BPW, _D), jnp.float32),
        pltpu.SemaphoreType.DMA,
    ],
)
def _gather_sc(uidx_hbm, iidx_hbm, utab_hbm, itab_hbm, u_out, i_out,
               idx_s, idx_v, rows_v, sem):
    wid = lax.axis_index("s") * _NC + lax.axis_index("c")
    base = wid * _BPW

    for tab_hbm, idx_hbm, out_hbm in (
        (utab_hbm, uidx_hbm, u_out),
        (itab_hbm, iidx_hbm, i_out),
    ):
        pltpu.sync_copy(idx_hbm.at[wid], idx_v)

        def row_body(g, carry, tab=tab_hbm):
            vec = idx_v[pl.ds(g * 16, 16)]
            for l in range(16):
                idx = vec[l]
                pltpu.async_copy(tab.at[pl.ds(idx, 1)],
                                 rows_v.at[pl.ds(g * 16 + l, 1)], sem)
            return carry

        lax.fori_loop(0, _BPW // 16, row_body, 0)
        pltpu.make_async_copy(tab_hbm.at[pl.ds(0, _BPW)], rows_v, sem).wait()
        pltpu.sync_copy(rows_v, out_hbm.at[pl.ds(base, _BPW)])


def _mlp_tc(u_ref, i_ref, w1_ref, b1_ref, w2_ref, b2_ref, o_ref):
    h = jnp.dot(u_ref[...], w1_ref[0:_D, :], preferred_element_type=jnp.float32)
    h = h + jnp.dot(i_ref[...], w1_ref[_D:, :],
                    preferred_element_type=jnp.float32)
    h = jnp.maximum(h + b1_ref[...], 0.0)
    logits = jnp.dot(h, w2_ref[...], preferred_element_type=jnp.float32)
    o_ref[...] = jax.nn.sigmoid(logits + b2_ref[...])


_BLK = 2048


def kernel(user_indices, item_indices, user_table, item_table, W1, b1, W2, b2):
    uidx = user_indices.astype(jnp.int32).reshape(_NW, _BPW)
    iidx = item_indices.astype(jnp.int32).reshape(_NW, _BPW)
    u_emb, i_emb = _gather_sc(uidx, iidx, user_table, item_table)

    out = pl.pallas_call(
        _mlp_tc,
        grid=(_B // _BLK,),
        in_specs=[
            pl.BlockSpec((_BLK, _D), lambda b: (b, 0)),
            pl.BlockSpec((_BLK, _D), lambda b: (b, 0)),
            pl.BlockSpec((2 * _D, _H), lambda b: (0, 0)),
            pl.BlockSpec((1, _H), lambda b: (0, 0)),
            pl.BlockSpec((_H, 1), lambda b: (0, 0)),
            pl.BlockSpec((1, 1), lambda b: (0, 0)),
        ],
        out_specs=pl.BlockSpec((_BLK, 1), lambda b: (b, 0)),
        out_shape=jax.ShapeDtypeStruct((_B, 1), jnp.float32),
    )(u_emb, i_emb, W1, b1.reshape(1, _H), W2, b2.reshape(1, 1))
    return out
